# baseline (speedup 1.0000x reference)
import functools

import jax
import jax.numpy as jnp
from jax import lax
from jax.experimental import pallas as pl
from jax.experimental.pallas import tpu as pltpu
from jax.experimental.pallas import tpu_sc as plsc

HID = 64
NG = 50176
NG2 = NG // 2
NM = 4096
NE = 200704
E2 = NE // 2
NV = 17
NSTEP = 2
NCORE, NSUB = 2, 16
NWORK = NCORE * NSUB


def _silu(x):
    return x * (1.0 / (1.0 + jnp.exp(-x)))


def _ln(x):
    mu = jnp.mean(x, axis=-1, keepdims=True)
    var = jnp.mean((x - mu) ** 2, axis=-1, keepdims=True)
    return (x - mu) * lax.rsqrt(var + 1e-5)


def _dot(a, b):
    return jnp.dot(a, b, preferred_element_type=jnp.float32)


_GW = 3584
_EB = 1024


def _rep(shape):
    return pl.BlockSpec(shape, lambda i: (0,) * len(shape))


def _grid_encode(xT, ge1, ge1b, ge2, ge2b, en1, en1b, en2, en2b, wsend1, wrec2):
    nb = NG2 // _GW

    def half(x, w1, b1, w2, b2, e1, eb1, e2, eb2, ws, wr):
        h = _silu(lax.dot_general(x, w1, (((0,), (0,)), ((), ())),
                                  preferred_element_type=jnp.float32) + b1)
        emb = _ln(_dot(h, w2) + b2)
        sp = _dot(emb, ws)
        g = _silu(_dot(emb, e1) + eb1)
        gr = emb + _ln(_dot(g, e2) + eb2)
        return sp, gr, _dot(gr, wr)

    def body(xlo_ref, xhi_ref, w1, b1, w2, b2, e1, eb1, e2, eb2, ws, wr,
             sp_ref, gr_ref, rp_ref):
        args = (w1[...], b1[...], w2[...], b2[...], e1[...], eb1[...],
                e2[...], eb2[...], ws[...], wr[...])
        sp, gr, rp = half(xlo_ref[...], *args)
        sp_ref[:, :HID] = sp
        gr_ref[:, :HID] = gr
        rp_ref[:, :HID] = rp
        sp, gr, rp = half(xhi_ref[...], *args)
        sp_ref[:, HID:] = sp
        gr_ref[:, HID:] = gr
        rp_ref[:, HID:] = rp

    return pl.pallas_call(
        body,
        grid=(nb,),
        in_specs=[
            pl.BlockSpec((NSTEP * NV, _GW), lambda i: (0, i)),
            pl.BlockSpec((NSTEP * NV, _GW), lambda i: (0, i + nb)),
            _rep((NSTEP * NV, HID)), _rep((1, HID)),
            _rep((HID, HID)), _rep((1, HID)),
            _rep((HID, HID)), _rep((1, HID)),
            _rep((HID, HID)), _rep((1, HID)),
            _rep((HID, HID)), _rep((HID, HID)),
        ],
        out_specs=[
            pl.BlockSpec((_GW, 2 * HID), lambda i: (i, 0)),
            pl.BlockSpec((_GW, 2 * HID), lambda i: (i, 0)),
            pl.BlockSpec((_GW, 2 * HID), lambda i: (i, 0)),
        ],
        out_shape=[jax.ShapeDtypeStruct((NG2, 2 * HID), jnp.float32)] * 3,
    )(xT, xT, ge1, ge1b, ge2, ge2b, en1, en1b, en2, en2b, wsend1, wrec2)


def _mesh_encode(msf, me1, me1b, me2, me2b, wrec1):
    def body(x_ref, w1, b1, w2, b2, wr, me_ref, rp_ref):
        h = _silu(_dot(x_ref[...], w1[...]) + b1[...])
        emb = _ln(_dot(h, w2[...]) + b2[...])
        me_ref[...] = emb
        rp_ref[...] = _dot(emb, wr[...])

    return pl.pallas_call(
        body,
        out_shape=[jax.ShapeDtypeStruct((NM, HID), jnp.float32)] * 2,
    )(msf, me1, me1b, me2, me2b, wrec1)


def _bd(w):
    z = jnp.zeros_like(w)
    return jnp.concatenate([jnp.concatenate([w, z], axis=1),
                            jnp.concatenate([z, w], axis=1)], axis=0)


def _tile2(b):
    return jnp.concatenate([b, b]).reshape(1, 2 * HID)


def _ln_half(x, mavg):
    mu = _dot(x, mavg)
    d = x - mu
    var = _dot(d * d, mavg)
    return d * lax.rsqrt(var + 1e-5)


def _edge_msgs(efT, gsum_p, em1, em1b, em2, em2b, w1e, b1, w2, b2):
    nb = E2 // _EB
    fdim = efT.shape[0]

    a1l = jnp.concatenate([em1, jnp.zeros_like(em1)], axis=1)
    a1h = jnp.concatenate([jnp.zeros_like(em1), em1], axis=1)
    a2bd, w1bd, w2bd = _bd(em2), _bd(w1e), _bd(w2)
    a1t, a2t, b1t, b2t = _tile2(em1b), _tile2(em2b), _tile2(b1), _tile2(b2)
    mavg = _bd(jnp.full((HID, HID), 1.0 / HID, jnp.float32))

    def body(flo_ref, fhi_ref, g_ref, a1l_r, a1h_r, a1t_r, a2_r, a2t_r,
             w1_r, b1_r, w2_r, b2_r, mavg_r, out_ref):
        cT = (((0,), (0,)), ((), ()))
        h = _silu(lax.dot_general(flo_ref[...], a1l_r[...], cT,
                                  preferred_element_type=jnp.float32)
                  + lax.dot_general(fhi_ref[...], a1h_r[...], cT,
                                    preferred_element_type=jnp.float32)
                  + a1t_r[...])
        emb = _ln_half(_dot(h, a2_r[...]) + a2t_r[...], mavg_r[...])
        m = _silu(_dot(emb, w1_r[...]) + b1_r[...] + g_ref[...])
        out_ref[...] = _ln_half(_dot(m, w2_r[...]) + b2_r[...], mavg_r[...])

    return pl.pallas_call(
        body,
        grid=(nb,),
        in_specs=[
            pl.BlockSpec((fdim, _EB), lambda i: (0, i)),
            pl.BlockSpec((fdim, _EB), lambda i: (0, i + nb)),
            pl.BlockSpec((_EB, 2 * HID), lambda i: (i, 0)),
            _rep((fdim, 2 * HID)), _rep((fdim, 2 * HID)), _rep((1, 2 * HID)),
            _rep((2 * HID, 2 * HID)), _rep((1, 2 * HID)),
            _rep((2 * HID, 2 * HID)), _rep((1, 2 * HID)),
            _rep((2 * HID, 2 * HID)), _rep((1, 2 * HID)),
            _rep((2 * HID, 2 * HID)),
        ],
        out_specs=pl.BlockSpec((_EB, 2 * HID), lambda i: (i, 0)),
        out_shape=jax.ShapeDtypeStruct((E2, 2 * HID), jnp.float32),
    )(efT, efT, gsum_p, a1l, a1h, a1t, a2bd, a2t, w1bd, b1t, w2bd, b2t, mavg)


def _mesh_update(me, partials_p, u1a, u1b_w, u1bias, u2, u2bias, wsend2):
    def body(me_ref, p_ref, wa, wb, bias1, w2, bias2, ws, sp_ref):
        aggr = p_ref[:, :HID] + p_ref[:, HID:]
        emb = me_ref[...]
        h = _silu(_dot(emb, wa[...]) + _dot(aggr, wb[...]) + bias1[...])
        mr = emb + _ln(_dot(h, w2[...]) + bias2[...])
        sp_ref[...] = _dot(mr, ws[...])

    return pl.pallas_call(
        body,
        out_shape=jax.ShapeDtypeStruct((NM, HID), jnp.float32),
    )(me, partials_p, u1a, u1b_w, u1bias, u2, u2bias, wsend2)


def _grid_update_out(gr_p, aggr_p, v1a, v1b_w, v1bias, v2, v2bias, o1, o1b, o2, o2b):
    nb = NG2 // _GW

    def half(g, a, wa, wb, bias1, w2, bias2, p1, pb1, p2, pb2):
        h = _silu(_dot(g, wa) + _dot(a, wb) + bias1)
        gnew = g + _ln(_dot(h, w2) + bias2)
        t = _silu(_dot(gnew, p1) + pb1)
        return _dot(t, p2) + pb2

    def body(g_ref, a_ref, wa, wb, bias1, w2, bias2, p1, pb1, p2, pb2,
             out_ref):
        args = (wa[...], wb[...], bias1[...], w2[...], bias2[...],
                p1[...], pb1[...], p2[...], pb2[...])
        out_ref[:, :NV] = half(g_ref[:, :HID], a_ref[:, :HID], *args)
        out_ref[:, NV:] = half(g_ref[:, HID:], a_ref[:, HID:], *args)

    return pl.pallas_call(
        body,
        grid=(nb,),
        in_specs=[
            pl.BlockSpec((_GW, 2 * HID), lambda i: (i, 0)),
            pl.BlockSpec((_GW, 2 * HID), lambda i: (i, 0)),
            _rep((HID, HID)), _rep((HID, HID)), _rep((1, HID)),
            _rep((HID, HID)), _rep((1, HID)),
            _rep((HID, HID)), _rep((1, HID)),
            _rep((HID, NV)), _rep((1, NV)),
        ],
        out_specs=pl.BlockSpec((_GW, 2 * NV), lambda i: (i, 0)),
        out_shape=jax.ShapeDtypeStruct((NG2, 2 * NV), jnp.float32),
    )(gr_p, aggr_p, v1a, v1b_w, v1bias, v2, v2bias, o1, o1b, o2, o2b)


_CH = 112
_SMROWS = NM // NSUB


def _sc_mesh():
    return plsc.VectorSubcoreMesh(core_axis_name="c", subcore_axis_name="s")


_SC_PARAMS = pltpu.CompilerParams(use_tc_tiling_on_sc=False)


_PERW = E2 // NWORK


@functools.partial(
    pl.kernel,
    out_type=jax.ShapeDtypeStruct((E2, 2 * HID), jnp.float32),
    mesh=_sc_mesh(),
    scratch_types=[
        pltpu.VMEM_SHARED((NM, HID), jnp.float32),
        pltpu.VMEM((_CH,), jnp.int32),
        pltpu.VMEM((_CH,), jnp.int32),
        pltpu.VMEM((_CH,), jnp.int32),
        pltpu.VMEM((_CH,), jnp.int32),
        pltpu.VMEM((_CH, HID), jnp.float32),
        pltpu.VMEM((_CH, HID), jnp.float32),
        pltpu.VMEM((_CH, HID), jnp.float32),
        pltpu.VMEM((_CH, HID), jnp.float32),
    ],
    compiler_params=_SC_PARAMS,
)
def _sc_gather_sum(big_hbm, small_hbm, idx_big_hbm, idx_small_hbm, out_hbm,
                   stab, ib_lo, ib_hi, is_lo, is_hi,
                   rb_lo, rb_hi, rs_lo, rs_hi):
    cid = lax.axis_index("c")
    sid = lax.axis_index("s")
    wid = sid * NCORE + cid
    n_ch = _PERW // _CH
    base_w = wid * _PERW

    pltpu.sync_copy(small_hbm.at[pl.ds(sid * _SMROWS, _SMROWS)],
                    stab.at[pl.ds(sid * _SMROWS, _SMROWS)])
    plsc.subcore_barrier()

    @pl.loop(0, n_ch)
    def _chunk(c):
        base = base_w + c * _CH
        pltpu.sync_copy(idx_big_hbm.at[pl.ds(base, _CH)], ib_lo)
        pltpu.sync_copy(idx_big_hbm.at[pl.ds(E2 + base, _CH)], ib_hi)
        pltpu.sync_copy(idx_small_hbm.at[pl.ds(base, _CH)], is_lo)
        pltpu.sync_copy(idx_small_hbm.at[pl.ds(E2 + base, _CH)], is_hi)
        for j in range(_CH // 16):
            sl = pl.ds(j * 16, 16)
            v = ib_lo[sl]
            ib_lo[sl] = jnp.where(v >= NG2, 2 * (v - NG2) + 1, 2 * v)
            w = ib_hi[sl]
            ib_hi[sl] = jnp.where(w >= NG2, 2 * (w - NG2) + 1, 2 * w)
        pltpu.sync_copy(big_hbm.at[ib_lo], rb_lo)
        pltpu.sync_copy(big_hbm.at[ib_hi], rb_hi)
        pltpu.sync_copy(stab.at[is_lo], rs_lo)
        pltpu.sync_copy(stab.at[is_hi], rs_hi)

        @pl.loop(0, _CH)
        def _row(r):
            for l in range(HID // 16):
                sl = pl.ds(l * 16, 16)
                rb_lo[r, sl] = rb_lo[r, sl] + rs_lo[r, sl]
                rb_hi[r, sl] = rb_hi[r, sl] + rs_hi[r, sl]

        pltpu.sync_copy(rb_lo, out_hbm.at[pl.ds(base, _CH), pl.ds(0, HID)])
        pltpu.sync_copy(rb_hi, out_hbm.at[pl.ds(base, _CH), pl.ds(HID, HID)])


@functools.partial(
    pl.kernel,
    out_type=jax.ShapeDtypeStruct((NM, 2 * HID), jnp.float32),
    mesh=_sc_mesh(),
    scratch_types=[
        pltpu.VMEM_SHARED((NM, HID), jnp.float32),
        pltpu.VMEM((_CH,), jnp.int32),
        pltpu.VMEM((_CH,), jnp.int32),
        pltpu.VMEM((_CH, HID), jnp.float32),
        pltpu.VMEM((_CH, HID), jnp.float32),
        pltpu.VMEM((_SMROWS, HID), jnp.float32),
    ],
    compiler_params=_SC_PARAMS,
)
def _sc_scatter_mesh(msgs_hbm, dst_hbm, out_hbm, acc, idx_lo, idx_hi,
                     m_lo, m_hi, buf_v):
    cid = lax.axis_index("c")
    sid = lax.axis_index("s")
    per_w = E2 // NWORK
    n_ch = per_w // _CH

    zvec = jnp.zeros((16,), jnp.float32)

    @pl.loop(0, _SMROWS)
    def _fillz(r):
        for l in range(HID // 16):
            buf_v[r, pl.ds(l * 16, 16)] = zvec

    pltpu.sync_copy(buf_v, acc.at[pl.ds(sid * _SMROWS, _SMROWS)])
    plsc.subcore_barrier()

    base_t = (cid * NSUB + sid) * per_w

    @pl.loop(0, n_ch)
    def _chunk(c):
        base = base_t + c * _CH
        pltpu.sync_copy(dst_hbm.at[pl.ds(base, _CH)], idx_lo)
        pltpu.sync_copy(dst_hbm.at[pl.ds(E2 + base, _CH)], idx_hi)
        pltpu.sync_copy(msgs_hbm.at[pl.ds(base, _CH), pl.ds(0, HID)], m_lo)
        pltpu.sync_copy(msgs_hbm.at[pl.ds(base, _CH), pl.ds(HID, HID)], m_hi)
        pltpu.sync_copy(m_lo, acc.at[idx_lo], add=True)
        pltpu.sync_copy(m_hi, acc.at[idx_hi], add=True)

    plsc.subcore_barrier()
    pltpu.sync_copy(acc.at[pl.ds(sid * _SMROWS, _SMROWS)], buf_v)
    pltpu.sync_copy(buf_v, out_hbm.at[pl.ds(sid * _SMROWS, _SMROWS),
                                      pl.ds(cid * HID, HID)])


_GACC = 25312
_GZB = 224


@functools.partial(
    pl.kernel,
    out_type=jax.ShapeDtypeStruct((NG2, 2 * HID), jnp.float32),
    mesh=_sc_mesh(),
    scratch_types=[
        pltpu.VMEM_SHARED((_GACC, HID), jnp.float32),
        pltpu.VMEM((_CH,), jnp.int32),
        pltpu.VMEM((_CH,), jnp.int32),
        pltpu.VMEM((_CH, HID), jnp.float32),
        pltpu.VMEM((_CH, HID), jnp.float32),
        pltpu.VMEM((_GZB, HID), jnp.float32),
    ],
    compiler_params=_SC_PARAMS,
)
def _sc_scatter_grid(msgs_hbm, dst_hbm, out_hbm, acc, idx_lo, idx_hi,
                     m_lo, m_hi, buf_v):
    cid = lax.axis_index("c")
    sid = lax.axis_index("s")
    per_tile = E2 // NSUB
    n_ch = per_tile // _CH
    base_node = cid * NG2

    zvec = jnp.zeros((16,), jnp.float32)

    @pl.loop(0, _GZB)
    def _fillz(r):
        for l in range(HID // 16):
            buf_v[r, pl.ds(l * 16, 16)] = zvec

    @pl.loop(sid, _GACC // _GZB, step=NSUB)
    def _zero(cblk):
        pltpu.sync_copy(buf_v, acc.at[pl.ds(cblk * _GZB, _GZB)])

    plsc.subcore_barrier()

    base_t = sid * per_tile

    @pl.loop(0, n_ch)
    def _chunk(c):
        base = base_t + c * _CH
        pltpu.sync_copy(dst_hbm.at[pl.ds(base, _CH)], idx_lo)
        pltpu.sync_copy(dst_hbm.at[pl.ds(E2 + base, _CH)], idx_hi)
        for j in range(_CH // 16):
            sl = pl.ds(j * 16, 16)
            v = idx_lo[sl] - base_node
            idx_lo[sl] = jnp.where((v >= 0) & (v < NG2), v, NG2)
            w = idx_hi[sl] - base_node
            idx_hi[sl] = jnp.where((w >= 0) & (w < NG2), w, NG2)
        pltpu.sync_copy(msgs_hbm.at[pl.ds(base, _CH), pl.ds(0, HID)], m_lo)
        pltpu.sync_copy(msgs_hbm.at[pl.ds(base, _CH), pl.ds(HID, HID)], m_hi)
        pltpu.sync_copy(m_lo, acc.at[idx_lo], add=True)
        pltpu.sync_copy(m_hi, acc.at[idx_hi], add=True)

    plsc.subcore_barrier()

    n_out = NG2 // NSUB // _GZB

    @pl.loop(0, n_out)
    def _out(c):
        row = sid * (NG2 // NSUB) + c * _GZB
        pltpu.sync_copy(acc.at[pl.ds(row, _GZB)], buf_v)
        pltpu.sync_copy(buf_v, out_hbm.at[pl.ds(row, _GZB),
                                          pl.ds(cid * HID, HID)])


def _b2d(b):
    return b.reshape(1, -1)


def kernel(prev_states, g2m_features, m2g_features, mesh_static_features,
           params, g2m_edge_index, m2g_edge_index):
    p = params
    (ge1, ge1b), (ge2, ge2b) = p['grid_embedder']
    (me1, me1b), (me2, me2b) = p['mesh_embedder']
    (ee1, ee1b), (ee2, ee2b) = p['g2m_embedder']
    (fe1, fe1b), (fe2, fe2b) = p['m2g_embedder']
    (en1, en1b), (en2, en2b) = p['encoding_grid_mlp']
    (gw1, gw1b), (gw2, gw2b) = p['g2m_edge_mlp']
    (ga1, ga1b), (ga2, ga2b) = p['g2m_aggr_mlp']
    (mw1, mw1b), (mw2, mw2b) = p['m2g_edge_mlp']
    (ma1, ma1b), (ma2, ma2b) = p['m2g_aggr_mlp']
    (o1, o1b), (o2, o2b) = p['output_map']

    xT = prev_states.reshape(NSTEP * NV, NG)

    sp1_p, gr_p, rp2_p = _grid_encode(
        xT, ge1, _b2d(ge1b), ge2, _b2d(ge2b),
        en1, _b2d(en1b), en2, _b2d(en2b),
        gw1[HID:2 * HID], mw1[2 * HID:3 * HID])
    sp1 = sp1_p.reshape(NG, HID)
    rp2 = rp2_p.reshape(NG, HID)

    me, rp1 = _mesh_encode(
        mesh_static_features, me1, _b2d(me1b), me2, _b2d(me2b),
        gw1[2 * HID:3 * HID])

    src1 = g2m_edge_index[0]
    dst1 = g2m_edge_index[1]
    gsum1_p = _sc_gather_sum(sp1, rp1, src1, dst1)
    msgs1_p = _edge_msgs(g2m_features.T, gsum1_p,
                         ee1, ee1b, ee2, ee2b,
                         gw1[:HID], gw1b, gw2, gw2b)
    partials1_p = _sc_scatter_mesh(msgs1_p, dst1)
    sp2 = _mesh_update(me, partials1_p,
                       ga1[:HID], ga1[HID:], _b2d(ga1b), ga2, _b2d(ga2b),
                       mw1[HID:2 * HID])

    src2 = m2g_edge_index[0]
    dst2 = m2g_edge_index[1]
    gsum2_p = _sc_gather_sum(rp2, sp2, dst2, src2)
    msgs2_p = _edge_msgs(m2g_features.T, gsum2_p,
                         fe1, fe1b, fe2, fe2b,
                         mw1[:HID], mw1b, mw2, mw2b)
    aggr2_p = _sc_scatter_grid(msgs2_p, dst2)
    pred_p = _grid_update_out(gr_p, aggr2_p,
                              ma1[:HID], ma1[HID:], _b2d(ma1b), ma2, _b2d(ma2b),
                              o1, _b2d(o1b), o2, _b2d(o2b))

    pred = jnp.concatenate([pred_p[:, :NV], pred_p[:, NV:]], axis=0)
    return pred.T.reshape(1, NV, 256, 196)


# speedup vs baseline: 1.1229x; 1.1229x over previous
import functools

import jax
import jax.numpy as jnp
from jax import lax
from jax.experimental import pallas as pl
from jax.experimental.pallas import tpu as pltpu
from jax.experimental.pallas import tpu_sc as plsc

HID = 64
NG = 50176
NG2 = NG // 2
NM = 4096
NE = 200704
E2 = NE // 2
NV = 17
NSTEP = 2
NCORE, NSUB = 2, 16
NWORK = NCORE * NSUB


def _silu(x):
    return x * (1.0 / (1.0 + jnp.exp(-x)))


def _ln(x):
    mu = jnp.mean(x, axis=-1, keepdims=True)
    var = jnp.mean((x - mu) ** 2, axis=-1, keepdims=True)
    return (x - mu) * lax.rsqrt(var + 1e-5)


def _dot(a, b):
    return jnp.dot(a, b, preferred_element_type=jnp.float32)


_GW = 3584
_EB = 1024


def _rep(shape):
    return pl.BlockSpec(shape, lambda i: (0,) * len(shape))


def _grid_encode(xT, ge1, ge1b, ge2, ge2b, en1, en1b, en2, en2b, wsend1, wrec2):
    nb = NG2 // _GW

    def half(x, w1, b1, w2, b2, e1, eb1, e2, eb2, ws, wr):
        h = _silu(lax.dot_general(x, w1, (((0,), (0,)), ((), ())),
                                  preferred_element_type=jnp.float32) + b1)
        emb = _ln(_dot(h, w2) + b2)
        sp = _dot(emb, ws)
        g = _silu(_dot(emb, e1) + eb1)
        gr = emb + _ln(_dot(g, e2) + eb2)
        return sp, gr, _dot(gr, wr)

    def body(xlo_ref, xhi_ref, w1, b1, w2, b2, e1, eb1, e2, eb2, ws, wr,
             sp_ref, gr_ref, rp_ref):
        args = (w1[...], b1[...], w2[...], b2[...], e1[...], eb1[...],
                e2[...], eb2[...], ws[...], wr[...])
        sp, gr, rp = half(xlo_ref[...], *args)
        sp_ref[:, :HID] = sp
        gr_ref[:, :HID] = gr
        rp_ref[:, :HID] = rp
        sp, gr, rp = half(xhi_ref[...], *args)
        sp_ref[:, HID:] = sp
        gr_ref[:, HID:] = gr
        rp_ref[:, HID:] = rp

    return pl.pallas_call(
        body,
        grid=(nb,),
        in_specs=[
            pl.BlockSpec((NSTEP * NV, _GW), lambda i: (0, i)),
            pl.BlockSpec((NSTEP * NV, _GW), lambda i: (0, i + nb)),
            _rep((NSTEP * NV, HID)), _rep((1, HID)),
            _rep((HID, HID)), _rep((1, HID)),
            _rep((HID, HID)), _rep((1, HID)),
            _rep((HID, HID)), _rep((1, HID)),
            _rep((HID, HID)), _rep((HID, HID)),
        ],
        out_specs=[
            pl.BlockSpec((_GW, 2 * HID), lambda i: (i, 0)),
            pl.BlockSpec((_GW, 2 * HID), lambda i: (i, 0)),
            pl.BlockSpec((_GW, 2 * HID), lambda i: (i, 0)),
        ],
        out_shape=[jax.ShapeDtypeStruct((NG2, 2 * HID), jnp.float32)] * 3,
    )(xT, xT, ge1, ge1b, ge2, ge2b, en1, en1b, en2, en2b, wsend1, wrec2)


def _mesh_encode(msf, me1, me1b, me2, me2b, wrec1):
    def body(x_ref, w1, b1, w2, b2, wr, me_ref, rp_ref):
        h = _silu(_dot(x_ref[...], w1[...]) + b1[...])
        emb = _ln(_dot(h, w2[...]) + b2[...])
        me_ref[...] = emb
        rp_ref[...] = _dot(emb, wr[...])

    return pl.pallas_call(
        body,
        out_shape=[jax.ShapeDtypeStruct((NM, HID), jnp.float32)] * 2,
    )(msf, me1, me1b, me2, me2b, wrec1)


def _bd(w):
    z = jnp.zeros_like(w)
    return jnp.concatenate([jnp.concatenate([w, z], axis=1),
                            jnp.concatenate([z, w], axis=1)], axis=0)


def _tile2(b):
    return jnp.concatenate([b, b]).reshape(1, 2 * HID)


def _ln_half(x, mavg):
    mu = _dot(x, mavg)
    d = x - mu
    var = _dot(d * d, mavg)
    return d * lax.rsqrt(var + 1e-5)


def _edge_msgs(efT, gsum_p, em1, em1b, em2, em2b, w1e, b1, w2, b2):
    nb = E2 // _EB
    fdim = efT.shape[0]

    a1l = jnp.concatenate([em1, jnp.zeros_like(em1)], axis=1)
    a1h = jnp.concatenate([jnp.zeros_like(em1), em1], axis=1)
    a2bd, w1bd, w2bd = _bd(em2), _bd(w1e), _bd(w2)
    a1t, a2t, b1t, b2t = _tile2(em1b), _tile2(em2b), _tile2(b1), _tile2(b2)
    mavg = _bd(jnp.full((HID, HID), 1.0 / HID, jnp.float32))

    def body(flo_ref, fhi_ref, g_ref, a1l_r, a1h_r, a1t_r, a2_r, a2t_r,
             w1_r, b1_r, w2_r, b2_r, mavg_r, out_ref):
        cT = (((0,), (0,)), ((), ()))
        h = _silu(lax.dot_general(flo_ref[...], a1l_r[...], cT,
                                  preferred_element_type=jnp.float32)
                  + lax.dot_general(fhi_ref[...], a1h_r[...], cT,
                                    preferred_element_type=jnp.float32)
                  + a1t_r[...])
        emb = _ln_half(_dot(h, a2_r[...]) + a2t_r[...], mavg_r[...])
        m = _silu(_dot(emb, w1_r[...]) + b1_r[...] + g_ref[...])
        out_ref[...] = _ln_half(_dot(m, w2_r[...]) + b2_r[...], mavg_r[...])

    return pl.pallas_call(
        body,
        grid=(nb,),
        in_specs=[
            pl.BlockSpec((fdim, _EB), lambda i: (0, i)),
            pl.BlockSpec((fdim, _EB), lambda i: (0, i + nb)),
            pl.BlockSpec((_EB, 2 * HID), lambda i: (i, 0)),
            _rep((fdim, 2 * HID)), _rep((fdim, 2 * HID)), _rep((1, 2 * HID)),
            _rep((2 * HID, 2 * HID)), _rep((1, 2 * HID)),
            _rep((2 * HID, 2 * HID)), _rep((1, 2 * HID)),
            _rep((2 * HID, 2 * HID)), _rep((1, 2 * HID)),
            _rep((2 * HID, 2 * HID)),
        ],
        out_specs=pl.BlockSpec((_EB, 2 * HID), lambda i: (i, 0)),
        out_shape=jax.ShapeDtypeStruct((E2, 2 * HID), jnp.float32),
    )(efT, efT, gsum_p, a1l, a1h, a1t, a2bd, a2t, w1bd, b1t, w2bd, b2t, mavg)


def _mesh_update(me, partials_p, u1a, u1b_w, u1bias, u2, u2bias, wsend2):
    def body(me_ref, p_ref, wa, wb, bias1, w2, bias2, ws, sp_ref):
        aggr = p_ref[:, :HID] + p_ref[:, HID:]
        emb = me_ref[...]
        h = _silu(_dot(emb, wa[...]) + _dot(aggr, wb[...]) + bias1[...])
        mr = emb + _ln(_dot(h, w2[...]) + bias2[...])
        sp_ref[...] = _dot(mr, ws[...])

    return pl.pallas_call(
        body,
        out_shape=jax.ShapeDtypeStruct((NM, HID), jnp.float32),
    )(me, partials_p, u1a, u1b_w, u1bias, u2, u2bias, wsend2)


def _grid_update_out(gr_p, aggr_p, v1a, v1b_w, v1bias, v2, v2bias, o1, o1b, o2, o2b):
    nb = NG2 // _GW

    def half(g, a, wa, wb, bias1, w2, bias2, p1, pb1, p2, pb2):
        h = _silu(_dot(g, wa) + _dot(a, wb) + bias1)
        gnew = g + _ln(_dot(h, w2) + bias2)
        t = _silu(_dot(gnew, p1) + pb1)
        return _dot(t, p2) + pb2

    def body(g_ref, a_ref, wa, wb, bias1, w2, bias2, p1, pb1, p2, pb2,
             out_ref):
        args = (wa[...], wb[...], bias1[...], w2[...], bias2[...],
                p1[...], pb1[...], p2[...], pb2[...])
        out_ref[:, :NV] = half(g_ref[:, :HID], a_ref[:, :HID], *args)
        out_ref[:, NV:] = half(g_ref[:, HID:], a_ref[:, HID:], *args)

    return pl.pallas_call(
        body,
        grid=(nb,),
        in_specs=[
            pl.BlockSpec((_GW, 2 * HID), lambda i: (i, 0)),
            pl.BlockSpec((_GW, 2 * HID), lambda i: (i, 0)),
            _rep((HID, HID)), _rep((HID, HID)), _rep((1, HID)),
            _rep((HID, HID)), _rep((1, HID)),
            _rep((HID, HID)), _rep((1, HID)),
            _rep((HID, NV)), _rep((1, NV)),
        ],
        out_specs=pl.BlockSpec((_GW, 2 * NV), lambda i: (i, 0)),
        out_shape=jax.ShapeDtypeStruct((NG2, 2 * NV), jnp.float32),
    )(gr_p, aggr_p, v1a, v1b_w, v1bias, v2, v2bias, o1, o1b, o2, o2b)


_CH = 224
_CHS = 448
_CHG = 112
_SMROWS = NM // NSUB


def _sc_mesh():
    return plsc.VectorSubcoreMesh(core_axis_name="c", subcore_axis_name="s")


_SC_PARAMS = pltpu.CompilerParams(use_tc_tiling_on_sc=False)


_PERW = E2 // NWORK


@functools.partial(
    pl.kernel,
    out_type=jax.ShapeDtypeStruct((E2, 2 * HID), jnp.float32),
    mesh=_sc_mesh(),
    scratch_types=[
        pltpu.VMEM_SHARED((NM, HID), jnp.float32),
        pltpu.VMEM((_CH,), jnp.int32),
        pltpu.VMEM((_CH,), jnp.int32),
        pltpu.VMEM((_CH,), jnp.int32),
        pltpu.VMEM((_CH,), jnp.int32),
        pltpu.VMEM((_CH, HID), jnp.float32),
        pltpu.VMEM((_CH, HID), jnp.float32),
        pltpu.VMEM((_CH, HID), jnp.float32),
        pltpu.VMEM((_CH, HID), jnp.float32),
    ],
    compiler_params=_SC_PARAMS,
)
def _sc_gather_sum(big_hbm, small_hbm, idx_big_hbm, idx_small_hbm, out_hbm,
                   stab, ib_lo, ib_hi, is_lo, is_hi,
                   rb_lo, rb_hi, rs_lo, rs_hi):
    cid = lax.axis_index("c")
    sid = lax.axis_index("s")
    wid = sid * NCORE + cid
    n_ch = _PERW // _CH
    base_w = wid * _PERW

    pltpu.sync_copy(small_hbm.at[pl.ds(sid * _SMROWS, _SMROWS)],
                    stab.at[pl.ds(sid * _SMROWS, _SMROWS)])
    plsc.subcore_barrier()

    @pl.loop(0, n_ch)
    def _chunk(c):
        base = base_w + c * _CH
        pltpu.sync_copy(idx_big_hbm.at[pl.ds(base, _CH)], ib_lo)
        pltpu.sync_copy(idx_big_hbm.at[pl.ds(E2 + base, _CH)], ib_hi)
        pltpu.sync_copy(idx_small_hbm.at[pl.ds(base, _CH)], is_lo)
        pltpu.sync_copy(idx_small_hbm.at[pl.ds(E2 + base, _CH)], is_hi)
        for j in range(_CH // 16):
            sl = pl.ds(j * 16, 16)
            v = ib_lo[sl]
            ib_lo[sl] = jnp.where(v >= NG2, 2 * (v - NG2) + 1, 2 * v)
            w = ib_hi[sl]
            ib_hi[sl] = jnp.where(w >= NG2, 2 * (w - NG2) + 1, 2 * w)
        pltpu.sync_copy(big_hbm.at[ib_lo], rb_lo)
        pltpu.sync_copy(big_hbm.at[ib_hi], rb_hi)
        pltpu.sync_copy(stab.at[is_lo], rs_lo)
        pltpu.sync_copy(stab.at[is_hi], rs_hi)

        @pl.loop(0, _CH)
        def _row(r):
            for l in range(HID // 16):
                sl = pl.ds(l * 16, 16)
                rb_lo[r, sl] = rb_lo[r, sl] + rs_lo[r, sl]
                rb_hi[r, sl] = rb_hi[r, sl] + rs_hi[r, sl]

        pltpu.sync_copy(rb_lo, out_hbm.at[pl.ds(base, _CH), pl.ds(0, HID)])
        pltpu.sync_copy(rb_hi, out_hbm.at[pl.ds(base, _CH), pl.ds(HID, HID)])


@functools.partial(
    pl.kernel,
    out_type=jax.ShapeDtypeStruct((NM, 2 * HID), jnp.float32),
    mesh=_sc_mesh(),
    scratch_types=[
        pltpu.VMEM_SHARED((NM, HID), jnp.float32),
        pltpu.VMEM((_CHS,), jnp.int32),
        pltpu.VMEM((_CHS,), jnp.int32),
        pltpu.VMEM((_CHS, HID), jnp.float32),
        pltpu.VMEM((_CHS, HID), jnp.float32),
        pltpu.VMEM((_SMROWS, HID), jnp.float32),
    ],
    compiler_params=_SC_PARAMS,
)
def _sc_scatter_mesh(msgs_hbm, dst_hbm, out_hbm, acc, idx_lo, idx_hi,
                     m_lo, m_hi, buf_v):
    cid = lax.axis_index("c")
    sid = lax.axis_index("s")
    per_w = E2 // NWORK
    n_ch = per_w // _CHS

    zvec = jnp.zeros((16,), jnp.float32)

    @pl.loop(0, _SMROWS)
    def _fillz(r):
        for l in range(HID // 16):
            buf_v[r, pl.ds(l * 16, 16)] = zvec

    pltpu.sync_copy(buf_v, acc.at[pl.ds(sid * _SMROWS, _SMROWS)])
    plsc.subcore_barrier()

    base_t = (cid * NSUB + sid) * per_w

    @pl.loop(0, n_ch)
    def _chunk(c):
        base = base_t + c * _CHS
        pltpu.sync_copy(dst_hbm.at[pl.ds(base, _CHS)], idx_lo)
        pltpu.sync_copy(dst_hbm.at[pl.ds(E2 + base, _CHS)], idx_hi)
        pltpu.sync_copy(msgs_hbm.at[pl.ds(base, _CHS), pl.ds(0, HID)], m_lo)
        pltpu.sync_copy(msgs_hbm.at[pl.ds(base, _CHS), pl.ds(HID, HID)], m_hi)
        pltpu.sync_copy(m_lo, acc.at[idx_lo], add=True)
        pltpu.sync_copy(m_hi, acc.at[idx_hi], add=True)

    plsc.subcore_barrier()
    pltpu.sync_copy(acc.at[pl.ds(sid * _SMROWS, _SMROWS)], buf_v)
    pltpu.sync_copy(buf_v, out_hbm.at[pl.ds(sid * _SMROWS, _SMROWS),
                                      pl.ds(cid * HID, HID)])


_GACC = 25312
_GZB = 224


@functools.partial(
    pl.kernel,
    out_type=jax.ShapeDtypeStruct((NG2, 2 * HID), jnp.float32),
    mesh=_sc_mesh(),
    scratch_types=[
        pltpu.VMEM_SHARED((_GACC, HID), jnp.float32),
        pltpu.VMEM((_CHG,), jnp.int32),
        pltpu.VMEM((_CHG,), jnp.int32),
        pltpu.VMEM((_CHG, HID), jnp.float32),
        pltpu.VMEM((_CHG, HID), jnp.float32),
        pltpu.VMEM((_GZB, HID), jnp.float32),
    ],
    compiler_params=_SC_PARAMS,
)
def _sc_scatter_grid(msgs_hbm, dst_hbm, out_hbm, acc, idx_lo, idx_hi,
                     m_lo, m_hi, buf_v):
    cid = lax.axis_index("c")
    sid = lax.axis_index("s")
    per_tile = E2 // NSUB
    n_ch = per_tile // _CHG
    base_node = cid * NG2

    zvec = jnp.zeros((16,), jnp.float32)

    @pl.loop(0, _GZB)
    def _fillz(r):
        for l in range(HID // 16):
            buf_v[r, pl.ds(l * 16, 16)] = zvec

    @pl.loop(sid, _GACC // _GZB, step=NSUB)
    def _zero(cblk):
        pltpu.sync_copy(buf_v, acc.at[pl.ds(cblk * _GZB, _GZB)])

    plsc.subcore_barrier()

    base_t = sid * per_tile

    @pl.loop(0, n_ch)
    def _chunk(c):
        base = base_t + c * _CHG
        pltpu.sync_copy(dst_hbm.at[pl.ds(base, _CHG)], idx_lo)
        pltpu.sync_copy(dst_hbm.at[pl.ds(E2 + base, _CHG)], idx_hi)
        for j in range(_CHG // 16):
            sl = pl.ds(j * 16, 16)
            v = idx_lo[sl] - base_node
            idx_lo[sl] = jnp.where((v >= 0) & (v < NG2), v, NG2)
            w = idx_hi[sl] - base_node
            idx_hi[sl] = jnp.where((w >= 0) & (w < NG2), w, NG2)
        pltpu.sync_copy(msgs_hbm.at[pl.ds(base, _CHG), pl.ds(0, HID)], m_lo)
        pltpu.sync_copy(msgs_hbm.at[pl.ds(base, _CHG), pl.ds(HID, HID)], m_hi)
        pltpu.sync_copy(m_lo, acc.at[idx_lo], add=True)
        pltpu.sync_copy(m_hi, acc.at[idx_hi], add=True)

    plsc.subcore_barrier()

    n_out = NG2 // NSUB // _GZB

    @pl.loop(0, n_out)
    def _out(c):
        row = sid * (NG2 // NSUB) + c * _GZB
        pltpu.sync_copy(acc.at[pl.ds(row, _GZB)], buf_v)
        pltpu.sync_copy(buf_v, out_hbm.at[pl.ds(row, _GZB),
                                          pl.ds(cid * HID, HID)])


def _b2d(b):
    return b.reshape(1, -1)


def kernel(prev_states, g2m_features, m2g_features, mesh_static_features,
           params, g2m_edge_index, m2g_edge_index):
    p = params
    (ge1, ge1b), (ge2, ge2b) = p['grid_embedder']
    (me1, me1b), (me2, me2b) = p['mesh_embedder']
    (ee1, ee1b), (ee2, ee2b) = p['g2m_embedder']
    (fe1, fe1b), (fe2, fe2b) = p['m2g_embedder']
    (en1, en1b), (en2, en2b) = p['encoding_grid_mlp']
    (gw1, gw1b), (gw2, gw2b) = p['g2m_edge_mlp']
    (ga1, ga1b), (ga2, ga2b) = p['g2m_aggr_mlp']
    (mw1, mw1b), (mw2, mw2b) = p['m2g_edge_mlp']
    (ma1, ma1b), (ma2, ma2b) = p['m2g_aggr_mlp']
    (o1, o1b), (o2, o2b) = p['output_map']

    xT = prev_states.reshape(NSTEP * NV, NG)

    sp1_p, gr_p, rp2_p = _grid_encode(
        xT, ge1, _b2d(ge1b), ge2, _b2d(ge2b),
        en1, _b2d(en1b), en2, _b2d(en2b),
        gw1[HID:2 * HID], mw1[2 * HID:3 * HID])
    sp1 = sp1_p.reshape(NG, HID)
    rp2 = rp2_p.reshape(NG, HID)

    me, rp1 = _mesh_encode(
        mesh_static_features, me1, _b2d(me1b), me2, _b2d(me2b),
        gw1[2 * HID:3 * HID])

    src1 = g2m_edge_index[0]
    dst1 = g2m_edge_index[1]
    gsum1_p = _sc_gather_sum(sp1, rp1, src1, dst1)
    msgs1_p = _edge_msgs(g2m_features.T, gsum1_p,
                         ee1, ee1b, ee2, ee2b,
                         gw1[:HID], gw1b, gw2, gw2b)
    partials1_p = _sc_scatter_mesh(msgs1_p, dst1)
    sp2 = _mesh_update(me, partials1_p,
                       ga1[:HID], ga1[HID:], _b2d(ga1b), ga2, _b2d(ga2b),
                       mw1[HID:2 * HID])

    src2 = m2g_edge_index[0]
    dst2 = m2g_edge_index[1]
    gsum2_p = _sc_gather_sum(rp2, sp2, dst2, src2)
    msgs2_p = _edge_msgs(m2g_features.T, gsum2_p,
                         fe1, fe1b, fe2, fe2b,
                         mw1[:HID], mw1b, mw2, mw2b)
    aggr2_p = _sc_scatter_grid(msgs2_p, dst2)
    pred_p = _grid_update_out(gr_p, aggr2_p,
                              ma1[:HID], ma1[HID:], _b2d(ma1b), ma2, _b2d(ma2b),
                              o1, _b2d(o1b), o2, _b2d(o2b))

    pred = jnp.concatenate([pred_p[:, :NV], pred_p[:, NV:]], axis=0)
    return pred.T.reshape(1, NV, 256, 196)


# speedup vs baseline: 1.1723x; 1.0440x over previous
import functools

import jax
import jax.numpy as jnp
from jax import lax
from jax.experimental import pallas as pl
from jax.experimental.pallas import tpu as pltpu
from jax.experimental.pallas import tpu_sc as plsc

HID = 64
NG = 50176
NG2 = NG // 2
NM = 4096
NE = 200704
E2 = NE // 2
NV = 17
NSTEP = 2
NCORE, NSUB = 2, 16
NWORK = NCORE * NSUB


def _silu(x):
    return x * (1.0 / (1.0 + jnp.exp(-x)))


def _ln(x):
    mu = jnp.mean(x, axis=-1, keepdims=True)
    var = jnp.mean((x - mu) ** 2, axis=-1, keepdims=True)
    return (x - mu) * lax.rsqrt(var + 1e-5)


def _dot(a, b):
    return jnp.dot(a, b, preferred_element_type=jnp.float32)


_GW = 3584
_EB = 1024


def _rep(shape):
    return pl.BlockSpec(shape, lambda i: (0,) * len(shape))


def _grid_encode(xT, ge1, ge1b, ge2, ge2b, en1, en1b, en2, en2b, wsend1, wrec2):
    nb = NG2 // _GW

    def half(x, w1, b1, w2, b2, e1, eb1, e2, eb2, ws, wr):
        h = _silu(lax.dot_general(x, w1, (((0,), (0,)), ((), ())),
                                  preferred_element_type=jnp.float32) + b1)
        emb = _ln(_dot(h, w2) + b2)
        sp = _dot(emb, ws)
        g = _silu(_dot(emb, e1) + eb1)
        gr = emb + _ln(_dot(g, e2) + eb2)
        return sp, gr, _dot(gr, wr)

    def body(xlo_ref, xhi_ref, w1, b1, w2, b2, e1, eb1, e2, eb2, ws, wr,
             sp_ref, gr_ref, rp_ref):
        args = (w1[...], b1[...], w2[...], b2[...], e1[...], eb1[...],
                e2[...], eb2[...], ws[...], wr[...])
        sp, gr, rp = half(xlo_ref[...], *args)
        sp_ref[:, :HID] = sp
        gr_ref[:, :HID] = gr
        rp_ref[:, :HID] = rp
        sp, gr, rp = half(xhi_ref[...], *args)
        sp_ref[:, HID:] = sp
        gr_ref[:, HID:] = gr
        rp_ref[:, HID:] = rp

    return pl.pallas_call(
        body,
        grid=(nb,),
        in_specs=[
            pl.BlockSpec((NSTEP * NV, _GW), lambda i: (0, i)),
            pl.BlockSpec((NSTEP * NV, _GW), lambda i: (0, i + nb)),
            _rep((NSTEP * NV, HID)), _rep((1, HID)),
            _rep((HID, HID)), _rep((1, HID)),
            _rep((HID, HID)), _rep((1, HID)),
            _rep((HID, HID)), _rep((1, HID)),
            _rep((HID, HID)), _rep((HID, HID)),
        ],
        out_specs=[
            pl.BlockSpec((_GW, 2 * HID), lambda i: (i, 0)),
            pl.BlockSpec((_GW, 2 * HID), lambda i: (i, 0)),
            pl.BlockSpec((_GW, 2 * HID), lambda i: (i, 0)),
        ],
        out_shape=[jax.ShapeDtypeStruct((NG2, 2 * HID), jnp.float32)] * 3,
    )(xT, xT, ge1, ge1b, ge2, ge2b, en1, en1b, en2, en2b, wsend1, wrec2)


def _mesh_encode(msf, me1, me1b, me2, me2b, wrec1):
    def body(x_ref, w1, b1, w2, b2, wr, me_ref, rp_ref):
        h = _silu(_dot(x_ref[...], w1[...]) + b1[...])
        emb = _ln(_dot(h, w2[...]) + b2[...])
        me_ref[...] = emb
        rp_ref[...] = _dot(emb, wr[...])

    return pl.pallas_call(
        body,
        out_shape=[jax.ShapeDtypeStruct((NM, HID), jnp.float32)] * 2,
    )(msf, me1, me1b, me2, me2b, wrec1)


def _bd(w):
    z = jnp.zeros_like(w)
    return jnp.concatenate([jnp.concatenate([w, z], axis=1),
                            jnp.concatenate([z, w], axis=1)], axis=0)


def _tile2(b):
    return jnp.concatenate([b, b]).reshape(1, 2 * HID)


def _ln_half(x, mavg):
    mu = _dot(x, mavg)
    d = x - mu
    var = _dot(d * d, mavg)
    return d * lax.rsqrt(var + 1e-5)


def _edge_msgs(efT, gsum_p, em1, em1b, em2, em2b, w1e, b1, w2, b2):
    nb = E2 // _EB
    fdim = efT.shape[0]

    a1l = jnp.concatenate([em1, jnp.zeros_like(em1)], axis=1)
    a1h = jnp.concatenate([jnp.zeros_like(em1), em1], axis=1)
    a2bd, w1bd, w2bd = _bd(em2), _bd(w1e), _bd(w2)
    a1t, a2t, b1t, b2t = _tile2(em1b), _tile2(em2b), _tile2(b1), _tile2(b2)
    mavg = _bd(jnp.full((HID, HID), 1.0 / HID, jnp.float32))

    def body(flo_ref, fhi_ref, g_ref, a1l_r, a1h_r, a1t_r, a2_r, a2t_r,
             w1_r, b1_r, w2_r, b2_r, mavg_r, out_ref):
        cT = (((0,), (0,)), ((), ()))
        h = _silu(lax.dot_general(flo_ref[...], a1l_r[...], cT,
                                  preferred_element_type=jnp.float32)
                  + lax.dot_general(fhi_ref[...], a1h_r[...], cT,
                                    preferred_element_type=jnp.float32)
                  + a1t_r[...])
        emb = _ln_half(_dot(h, a2_r[...]) + a2t_r[...], mavg_r[...])
        m = _silu(_dot(emb, w1_r[...]) + b1_r[...] + g_ref[...])
        out_ref[...] = _ln_half(_dot(m, w2_r[...]) + b2_r[...], mavg_r[...])

    return pl.pallas_call(
        body,
        grid=(nb,),
        in_specs=[
            pl.BlockSpec((fdim, _EB), lambda i: (0, i)),
            pl.BlockSpec((fdim, _EB), lambda i: (0, i + nb)),
            pl.BlockSpec((_EB, 2 * HID), lambda i: (i, 0)),
            _rep((fdim, 2 * HID)), _rep((fdim, 2 * HID)), _rep((1, 2 * HID)),
            _rep((2 * HID, 2 * HID)), _rep((1, 2 * HID)),
            _rep((2 * HID, 2 * HID)), _rep((1, 2 * HID)),
            _rep((2 * HID, 2 * HID)), _rep((1, 2 * HID)),
            _rep((2 * HID, 2 * HID)),
        ],
        out_specs=pl.BlockSpec((_EB, 2 * HID), lambda i: (i, 0)),
        out_shape=jax.ShapeDtypeStruct((E2, 2 * HID), jnp.float32),
    )(efT, efT, gsum_p, a1l, a1h, a1t, a2bd, a2t, w1bd, b1t, w2bd, b2t, mavg)


def _mesh_update(me, partials_p, u1a, u1b_w, u1bias, u2, u2bias, wsend2):
    def body(me_ref, p_ref, wa, wb, bias1, w2, bias2, ws, sp_ref):
        aggr = p_ref[:, :HID] + p_ref[:, HID:]
        emb = me_ref[...]
        h = _silu(_dot(emb, wa[...]) + _dot(aggr, wb[...]) + bias1[...])
        mr = emb + _ln(_dot(h, w2[...]) + bias2[...])
        sp_ref[...] = _dot(mr, ws[...])

    return pl.pallas_call(
        body,
        out_shape=jax.ShapeDtypeStruct((NM, HID), jnp.float32),
    )(me, partials_p, u1a, u1b_w, u1bias, u2, u2bias, wsend2)


def _grid_update_out(gr_p, aggr_p, v1a, v1b_w, v1bias, v2, v2bias, o1, o1b, o2, o2b):
    nb = NG2 // _GW

    def half(g, a, wa, wb, bias1, w2, bias2, p1, pb1, p2, pb2):
        h = _silu(_dot(g, wa) + _dot(a, wb) + bias1)
        gnew = g + _ln(_dot(h, w2) + bias2)
        t = _silu(_dot(gnew, p1) + pb1)
        return _dot(t, p2) + pb2

    def body(g_ref, a_ref, wa, wb, bias1, w2, bias2, p1, pb1, p2, pb2,
             out_ref):
        args = (wa[...], wb[...], bias1[...], w2[...], bias2[...],
                p1[...], pb1[...], p2[...], pb2[...])
        out_ref[:, :NV] = half(g_ref[:, :HID], a_ref[:, :HID], *args)
        out_ref[:, NV:] = half(g_ref[:, HID:], a_ref[:, HID:], *args)

    return pl.pallas_call(
        body,
        grid=(nb,),
        in_specs=[
            pl.BlockSpec((_GW, 2 * HID), lambda i: (i, 0)),
            pl.BlockSpec((_GW, 2 * HID), lambda i: (i, 0)),
            _rep((HID, HID)), _rep((HID, HID)), _rep((1, HID)),
            _rep((HID, HID)), _rep((1, HID)),
            _rep((HID, HID)), _rep((1, HID)),
            _rep((HID, NV)), _rep((1, NV)),
        ],
        out_specs=pl.BlockSpec((_GW, 2 * NV), lambda i: (i, 0)),
        out_shape=jax.ShapeDtypeStruct((NG2, 2 * NV), jnp.float32),
    )(gr_p, aggr_p, v1a, v1b_w, v1bias, v2, v2bias, o1, o1b, o2, o2b)


_CH = 224
_CHS = 448
_CHG = 112
_SMROWS = NM // NSUB


def _sc_mesh():
    return plsc.VectorSubcoreMesh(core_axis_name="c", subcore_axis_name="s")


_SC_PARAMS = pltpu.CompilerParams(use_tc_tiling_on_sc=False)


_PERW = E2 // NWORK


@functools.partial(
    pl.kernel,
    out_type=jax.ShapeDtypeStruct((E2, 2 * HID), jnp.float32),
    mesh=_sc_mesh(),
    scratch_types=[
        pltpu.VMEM_SHARED((NM, HID), jnp.float32),
        pltpu.VMEM((_PERW,), jnp.int32),
        pltpu.VMEM((_PERW,), jnp.int32),
        pltpu.VMEM((_PERW,), jnp.int32),
        pltpu.VMEM((_PERW,), jnp.int32),
        pltpu.VMEM((_CH,), jnp.int32),
        pltpu.VMEM((_CH,), jnp.int32),
        pltpu.VMEM((_CH,), jnp.int32),
        pltpu.VMEM((_CH,), jnp.int32),
        pltpu.VMEM((_CH, HID), jnp.float32),
        pltpu.VMEM((_CH, HID), jnp.float32),
        pltpu.VMEM((_CH, HID), jnp.float32),
        pltpu.VMEM((_CH, HID), jnp.float32),
    ],
    compiler_params=_SC_PARAMS,
)
def _sc_gather_sum(big_hbm, small_hbm, idx_big_hbm, idx_small_hbm, out_hbm,
                   stab, ibp, ihp, jlp, jhp, ib_lo, ib_hi, is_lo, is_hi,
                   rb_lo, rb_hi, rs_lo, rs_hi):
    cid = lax.axis_index("c")
    sid = lax.axis_index("s")
    wid = sid * NCORE + cid
    n_ch = _PERW // _CH
    base_w = wid * _PERW

    pltpu.sync_copy(small_hbm.at[pl.ds(sid * _SMROWS, _SMROWS)],
                    stab.at[pl.ds(sid * _SMROWS, _SMROWS)])

    pltpu.sync_copy(idx_big_hbm.at[pl.ds(base_w, _PERW)], ibp)
    pltpu.sync_copy(idx_big_hbm.at[pl.ds(E2 + base_w, _PERW)], ihp)
    pltpu.sync_copy(idx_small_hbm.at[pl.ds(base_w, _PERW)], jlp)
    pltpu.sync_copy(idx_small_hbm.at[pl.ds(E2 + base_w, _PERW)], jhp)

    @pl.loop(0, _PERW // 16)
    def _rho(j):
        sl = pl.ds(j * 16, 16)
        v = ibp[sl]
        ibp[sl] = jnp.where(v >= NG2, 2 * (v - NG2) + 1, 2 * v)
        w = ihp[sl]
        ihp[sl] = jnp.where(w >= NG2, 2 * (w - NG2) + 1, 2 * w)

    plsc.subcore_barrier()

    @pl.loop(0, n_ch)
    def _chunk(c):
        base = base_w + c * _CH
        off = c * _CH
        for j in range(_CH // 16):
            d = pl.ds(j * 16, 16)
            src = pl.ds(off + j * 16, 16)
            ib_lo[d] = ibp[src]
            ib_hi[d] = ihp[src]
            is_lo[d] = jlp[src]
            is_hi[d] = jhp[src]
        pltpu.sync_copy(big_hbm.at[ib_lo], rb_lo)
        pltpu.sync_copy(big_hbm.at[ib_hi], rb_hi)
        pltpu.sync_copy(stab.at[is_lo], rs_lo)
        pltpu.sync_copy(stab.at[is_hi], rs_hi)

        @pl.loop(0, _CH)
        def _row(r):
            for l in range(HID // 16):
                sl = pl.ds(l * 16, 16)
                rb_lo[r, sl] = rb_lo[r, sl] + rs_lo[r, sl]
                rb_hi[r, sl] = rb_hi[r, sl] + rs_hi[r, sl]

        pltpu.sync_copy(rb_lo, out_hbm.at[pl.ds(base, _CH), pl.ds(0, HID)])
        pltpu.sync_copy(rb_hi, out_hbm.at[pl.ds(base, _CH), pl.ds(HID, HID)])


@functools.partial(
    pl.kernel,
    out_type=jax.ShapeDtypeStruct((NM, 2 * HID), jnp.float32),
    mesh=_sc_mesh(),
    scratch_types=[
        pltpu.VMEM_SHARED((NM, HID), jnp.float32),
        pltpu.VMEM((_CHS,), jnp.int32),
        pltpu.VMEM((_CHS,), jnp.int32),
        pltpu.VMEM((_CHS, HID), jnp.float32),
        pltpu.VMEM((_CHS, HID), jnp.float32),
        pltpu.VMEM((_SMROWS, HID), jnp.float32),
    ],
    compiler_params=_SC_PARAMS,
)
def _sc_scatter_mesh(msgs_hbm, dst_hbm, out_hbm, acc, idx_lo, idx_hi,
                     m_lo, m_hi, buf_v):
    cid = lax.axis_index("c")
    sid = lax.axis_index("s")
    per_w = E2 // NWORK
    n_ch = per_w // _CHS

    zvec = jnp.zeros((16,), jnp.float32)

    @pl.loop(0, _SMROWS)
    def _fillz(r):
        for l in range(HID // 16):
            buf_v[r, pl.ds(l * 16, 16)] = zvec

    pltpu.sync_copy(buf_v, acc.at[pl.ds(sid * _SMROWS, _SMROWS)])
    plsc.subcore_barrier()

    base_t = (cid * NSUB + sid) * per_w

    @pl.loop(0, n_ch)
    def _chunk(c):
        base = base_t + c * _CHS
        pltpu.sync_copy(dst_hbm.at[pl.ds(base, _CHS)], idx_lo)
        pltpu.sync_copy(dst_hbm.at[pl.ds(E2 + base, _CHS)], idx_hi)
        pltpu.sync_copy(msgs_hbm.at[pl.ds(base, _CHS), pl.ds(0, HID)], m_lo)
        pltpu.sync_copy(msgs_hbm.at[pl.ds(base, _CHS), pl.ds(HID, HID)], m_hi)
        pltpu.sync_copy(m_lo, acc.at[idx_lo], add=True)
        pltpu.sync_copy(m_hi, acc.at[idx_hi], add=True)

    plsc.subcore_barrier()
    pltpu.sync_copy(acc.at[pl.ds(sid * _SMROWS, _SMROWS)], buf_v)
    pltpu.sync_copy(buf_v, out_hbm.at[pl.ds(sid * _SMROWS, _SMROWS),
                                      pl.ds(cid * HID, HID)])


_GACC = 25312
_GZB = 224


@functools.partial(
    pl.kernel,
    out_type=jax.ShapeDtypeStruct((NG2, 2 * HID), jnp.float32),
    mesh=_sc_mesh(),
    scratch_types=[
        pltpu.VMEM_SHARED((_GACC, HID), jnp.float32),
        pltpu.VMEM((_CHG,), jnp.int32),
        pltpu.VMEM((_CHG,), jnp.int32),
        pltpu.VMEM((_CHG, HID), jnp.float32),
        pltpu.VMEM((_CHG, HID), jnp.float32),
        pltpu.VMEM((_GZB, HID), jnp.float32),
    ],
    compiler_params=_SC_PARAMS,
)
def _sc_scatter_grid(msgs_hbm, dst_hbm, out_hbm, acc, idx_lo, idx_hi,
                     m_lo, m_hi, buf_v):
    cid = lax.axis_index("c")
    sid = lax.axis_index("s")
    per_tile = E2 // NSUB
    n_ch = per_tile // _CHG
    base_node = cid * NG2

    zvec = jnp.zeros((16,), jnp.float32)

    @pl.loop(0, _GZB)
    def _fillz(r):
        for l in range(HID // 16):
            buf_v[r, pl.ds(l * 16, 16)] = zvec

    @pl.loop(sid, _GACC // _GZB, step=NSUB)
    def _zero(cblk):
        pltpu.sync_copy(buf_v, acc.at[pl.ds(cblk * _GZB, _GZB)])

    plsc.subcore_barrier()

    base_t = sid * per_tile

    @pl.loop(0, n_ch)
    def _chunk(c):
        base = base_t + c * _CHG
        pltpu.sync_copy(dst_hbm.at[pl.ds(base, _CHG)], idx_lo)
        pltpu.sync_copy(dst_hbm.at[pl.ds(E2 + base, _CHG)], idx_hi)
        for j in range(_CHG // 16):
            sl = pl.ds(j * 16, 16)
            v = idx_lo[sl] - base_node
            idx_lo[sl] = jnp.where((v >= 0) & (v < NG2), v, NG2)
            w = idx_hi[sl] - base_node
            idx_hi[sl] = jnp.where((w >= 0) & (w < NG2), w, NG2)
        pltpu.sync_copy(msgs_hbm.at[pl.ds(base, _CHG), pl.ds(0, HID)], m_lo)
        pltpu.sync_copy(msgs_hbm.at[pl.ds(base, _CHG), pl.ds(HID, HID)], m_hi)
        pltpu.sync_copy(m_lo, acc.at[idx_lo], add=True)
        pltpu.sync_copy(m_hi, acc.at[idx_hi], add=True)

    plsc.subcore_barrier()

    n_out = NG2 // NSUB // _GZB

    @pl.loop(0, n_out)
    def _out(c):
        row = sid * (NG2 // NSUB) + c * _GZB
        pltpu.sync_copy(acc.at[pl.ds(row, _GZB)], buf_v)
        pltpu.sync_copy(buf_v, out_hbm.at[pl.ds(row, _GZB),
                                          pl.ds(cid * HID, HID)])


def _b2d(b):
    return b.reshape(1, -1)


def kernel(prev_states, g2m_features, m2g_features, mesh_static_features,
           params, g2m_edge_index, m2g_edge_index):
    p = params
    (ge1, ge1b), (ge2, ge2b) = p['grid_embedder']
    (me1, me1b), (me2, me2b) = p['mesh_embedder']
    (ee1, ee1b), (ee2, ee2b) = p['g2m_embedder']
    (fe1, fe1b), (fe2, fe2b) = p['m2g_embedder']
    (en1, en1b), (en2, en2b) = p['encoding_grid_mlp']
    (gw1, gw1b), (gw2, gw2b) = p['g2m_edge_mlp']
    (ga1, ga1b), (ga2, ga2b) = p['g2m_aggr_mlp']
    (mw1, mw1b), (mw2, mw2b) = p['m2g_edge_mlp']
    (ma1, ma1b), (ma2, ma2b) = p['m2g_aggr_mlp']
    (o1, o1b), (o2, o2b) = p['output_map']

    xT = prev_states.reshape(NSTEP * NV, NG)

    sp1_p, gr_p, rp2_p = _grid_encode(
        xT, ge1, _b2d(ge1b), ge2, _b2d(ge2b),
        en1, _b2d(en1b), en2, _b2d(en2b),
        gw1[HID:2 * HID], mw1[2 * HID:3 * HID])
    sp1 = sp1_p.reshape(NG, HID)
    rp2 = rp2_p.reshape(NG, HID)

    me, rp1 = _mesh_encode(
        mesh_static_features, me1, _b2d(me1b), me2, _b2d(me2b),
        gw1[2 * HID:3 * HID])

    src1 = g2m_edge_index[0]
    dst1 = g2m_edge_index[1]
    gsum1_p = _sc_gather_sum(sp1, rp1, src1, dst1)
    msgs1_p = _edge_msgs(g2m_features.T, gsum1_p,
                         ee1, ee1b, ee2, ee2b,
                         gw1[:HID], gw1b, gw2, gw2b)
    partials1_p = _sc_scatter_mesh(msgs1_p, dst1)
    sp2 = _mesh_update(me, partials1_p,
                       ga1[:HID], ga1[HID:], _b2d(ga1b), ga2, _b2d(ga2b),
                       mw1[HID:2 * HID])

    src2 = m2g_edge_index[0]
    dst2 = m2g_edge_index[1]
    gsum2_p = _sc_gather_sum(rp2, sp2, dst2, src2)
    msgs2_p = _edge_msgs(m2g_features.T, gsum2_p,
                         fe1, fe1b, fe2, fe2b,
                         mw1[:HID], mw1b, mw2, mw2b)
    aggr2_p = _sc_scatter_grid(msgs2_p, dst2)
    pred_p = _grid_update_out(gr_p, aggr2_p,
                              ma1[:HID], ma1[HID:], _b2d(ma1b), ma2, _b2d(ma2b),
                              o1, _b2d(o1b), o2, _b2d(o2b))

    pred = jnp.concatenate([pred_p[:, :NV], pred_p[:, NV:]], axis=0)
    return pred.T.reshape(1, NV, 256, 196)
